# baseline (speedup 1.0000x reference)
import functools

import jax
import jax.numpy as jnp
from jax import lax
from jax.experimental import pallas as pl
from jax.experimental.pallas import tpu as pltpu
from jax.experimental.pallas import tpu_sc as plsc

B_ROWS, SEQ = 4096, 192
D = 48
DPAD = 128
NUM_CORES, NUM_SUBCORES = 2, 16
NW = NUM_CORES * NUM_SUBCORES
IPW = B_ROWS // NW
L = 16

NBUF = 4
GA = 3
MBUF = 4

_mesh = plsc.VectorSubcoreMesh(core_axis_name="c", subcore_axis_name="s")


@functools.partial(
    pl.kernel,
    out_type=jax.ShapeDtypeStruct((SEQ, D, B_ROWS), jnp.float32),
    mesh=_mesh,
    scratch_types=[
        pltpu.VMEM((SEQ, IPW), jnp.int32),
        pltpu.VMEM((NBUF, IPW, DPAD), jnp.float32),
        pltpu.VMEM((MBUF, D, IPW), jnp.float32),
        pltpu.SemaphoreType.DMA,
        pltpu.SemaphoreType.DMA,
    ],
    compiler_params=pltpu.CompilerParams(needs_layout_passes=False),
)
def _gather_t_kernel(table_hbm, idxt_hbm, out_hbm, idx_v, raw, stg, gsem, wsem):
    wid = lax.axis_index("s") * NUM_CORES + lax.axis_index("c")
    i0 = wid * IPW
    pltpu.sync_copy(idxt_hbm.at[wid], idx_v)

    lane = lax.iota(jnp.int32, L)
    row_ids = [lane + (v * L) for v in range(IPW // L)]

    def start_gather(j, b):
        pltpu.async_copy(table_hbm.at[idx_v.at[j]], raw.at[b], gsem)

    for b in range(GA):
        start_gather(b, b)

    @pl.loop(0, SEQ, step=NBUF)
    def _(j0):
        for b in range(NBUF):
            j = j0 + b
            pltpu.make_async_copy(
                table_hbm.at[pl.ds(0, IPW)], raw.at[b], gsem
            ).wait()

            @pl.when(j + GA < SEQ)
            def _():
                start_gather(j + GA, (b + GA) % NBUF)

            @pl.when(j >= MBUF)
            def _():
                pltpu.make_async_copy(
                    stg.at[b], out_hbm.at[0, :, pl.ds(0, IPW)], wsem
                ).wait()

            @pl.loop(0, L, unroll=8)
            def _(d):
                diag = lax.bitwise_and(lane + d, jnp.full((L,), L - 1, jnp.int32))
                for v in range(IPW // L):
                    for u in range(D // L):
                        cols = diag + (u * L)
                        vec = plsc.load_gather(raw.at[b], [row_ids[v], cols])
                        plsc.store_scatter(stg.at[b], [cols, row_ids[v]], vec)

            pltpu.async_copy(stg.at[b], out_hbm.at[j, :, pl.ds(i0, IPW)], wsem)

    for _ in range(MBUF):
        pltpu.make_async_copy(
            stg.at[0], out_hbm.at[0, :, pl.ds(0, IPW)], wsem
        ).wait()


def kernel(camera_indices, table):
    table_p = jnp.pad(table, ((0, 0), (0, DPAD - D)))
    idx_blk = (
        camera_indices.astype(jnp.int32)
        .reshape(NW, IPW, SEQ)
        .transpose(0, 2, 1)
    )
    out_t = _gather_t_kernel(table_p, idx_blk)
    return out_t.transpose(2, 0, 1)


# speedup vs baseline: 1.4421x; 1.4421x over previous
import functools

import jax
import jax.numpy as jnp
from jax import lax
from jax.experimental import pallas as pl
from jax.experimental.pallas import tpu as pltpu
from jax.experimental.pallas import tpu_sc as plsc

B_ROWS, SEQ = 4096, 192
D = 48
DPAD = 128
NUM_CORES, NUM_SUBCORES = 2, 16
NW = NUM_CORES * NUM_SUBCORES
IPW = B_ROWS // NW
L = 16

NBUF = 4
GA = 3
MBUF = 4

_mesh = plsc.VectorSubcoreMesh(core_axis_name="c", subcore_axis_name="s")


@functools.partial(
    pl.kernel,
    out_type=jax.ShapeDtypeStruct((SEQ, D, B_ROWS), jnp.float32),
    mesh=_mesh,
    scratch_types=[
        pltpu.VMEM((SEQ, IPW), jnp.int32),
        pltpu.VMEM((NBUF, IPW, DPAD), jnp.float32),
        pltpu.VMEM((MBUF, D, IPW), jnp.float32),
        pltpu.SemaphoreType.DMA,
        pltpu.SemaphoreType.DMA,
    ],
    compiler_params=pltpu.CompilerParams(needs_layout_passes=False),
)
def _gather_t_kernel(table_hbm, idxt_hbm, out_hbm, idx_v, raw, stg, gsem, wsem):
    wid = lax.axis_index("s") * NUM_CORES + lax.axis_index("c")
    i0 = wid * IPW
    pltpu.sync_copy(idxt_hbm.at[wid], idx_v)

    lane = lax.iota(jnp.int32, L)
    row_ids = [lane + (v * L) for v in range(IPW // L)]

    def start_gather(j, b):
        pltpu.async_copy(table_hbm.at[idx_v.at[j]], raw.at[b], gsem)

    for b in range(GA):
        start_gather(b, b)

    @pl.loop(0, SEQ, step=NBUF)
    def _(j0):
        for b in range(NBUF):
            j = j0 + b
            pltpu.make_async_copy(
                table_hbm.at[pl.ds(0, IPW)], raw.at[b], gsem
            ).wait()

            @pl.when(j + GA < SEQ)
            def _():
                start_gather(j + GA, (b + GA) % NBUF)

            @pl.when(j >= MBUF)
            def _():
                pltpu.make_async_copy(
                    stg.at[b], out_hbm.at[0, :, pl.ds(0, IPW)], wsem
                ).wait()

            @pl.loop(0, L, unroll=4)
            def _(d):
                diag = lax.bitwise_and(lane + d, jnp.full((L,), L - 1, jnp.int32))
                colsets = [diag + (u * L) for u in range(D // L)]
                vecs = [
                    plsc.load_gather(raw.at[b], [row_ids[v], colsets[u]])
                    for v in range(IPW // L)
                    for u in range(D // L)
                ]
                n = 0
                for v in range(IPW // L):
                    for u in range(D // L):
                        plsc.store_scatter(
                            stg.at[b], [colsets[u], row_ids[v]], vecs[n]
                        )
                        n += 1

            pltpu.async_copy(stg.at[b], out_hbm.at[j, :, pl.ds(i0, IPW)], wsem)

    for _ in range(MBUF):
        pltpu.make_async_copy(
            stg.at[0], out_hbm.at[0, :, pl.ds(0, IPW)], wsem
        ).wait()


def kernel(camera_indices, table):
    table_p = jnp.pad(table, ((0, 0), (0, DPAD - D)))
    idx_blk = (
        camera_indices.astype(jnp.int32)
        .reshape(NW, IPW, SEQ)
        .transpose(0, 2, 1)
    )
    out_t = _gather_t_kernel(table_p, idx_blk)
    return out_t.transpose(2, 0, 1)


# speedup vs baseline: 1.4445x; 1.0017x over previous
import functools

import jax
import jax.numpy as jnp
from jax import lax
from jax.experimental import pallas as pl
from jax.experimental.pallas import tpu as pltpu
from jax.experimental.pallas import tpu_sc as plsc

B_ROWS, SEQ = 4096, 192
D = 48
DPAD = 128
NUM_CORES, NUM_SUBCORES = 2, 16
NW = NUM_CORES * NUM_SUBCORES
IPW = B_ROWS // NW
L = 16

NBUF = 4
GA = 3
MBUF = 4

_mesh = plsc.VectorSubcoreMesh(core_axis_name="c", subcore_axis_name="s")


@functools.partial(
    pl.kernel,
    out_type=jax.ShapeDtypeStruct((SEQ, D, B_ROWS), jnp.float32),
    mesh=_mesh,
    scratch_types=[
        pltpu.VMEM((SEQ, IPW), jnp.int32),
        pltpu.VMEM((NBUF, IPW, DPAD), jnp.float32),
        pltpu.VMEM((MBUF, D, IPW), jnp.float32),
        pltpu.SemaphoreType.DMA,
        pltpu.SemaphoreType.DMA,
    ],
    compiler_params=pltpu.CompilerParams(needs_layout_passes=False),
)
def _gather_t_kernel(table_hbm, idxt_hbm, out_hbm, idx_v, raw, stg, gsem, wsem):
    wid = lax.axis_index("s") * NUM_CORES + lax.axis_index("c")
    i0 = wid * IPW
    pltpu.sync_copy(idxt_hbm.at[wid], idx_v)

    lane = lax.iota(jnp.int32, L)
    row_ids = [lane + (v * L) for v in range(IPW // L)]

    def start_gather(j, b):
        pltpu.async_copy(table_hbm.at[idx_v.at[j]], raw.at[b], gsem)

    for b in range(GA):
        start_gather(b, b)

    @pl.loop(0, SEQ, step=NBUF)
    def _(j0):
        for b in range(NBUF):
            j = j0 + b
            pltpu.make_async_copy(
                table_hbm.at[pl.ds(0, IPW)], raw.at[b], gsem
            ).wait()

            @pl.when(j + GA < SEQ)
            def _():
                start_gather(j + GA, (b + GA) % NBUF)

            @pl.when(j >= MBUF)
            def _():
                pltpu.make_async_copy(
                    stg.at[b], out_hbm.at[0, :, pl.ds(0, IPW)], wsem
                ).wait()

            @pl.loop(0, L, unroll=2)
            def _(d):
                diag = lax.bitwise_and(lane + d, jnp.full((L,), L - 1, jnp.int32))
                colsets = [diag + (u * L) for u in range(D // L)]
                vecs = [
                    plsc.load_gather(raw.at[b], [row_ids[v], colsets[u]])
                    for v in range(IPW // L)
                    for u in range(D // L)
                ]
                n = 0
                for v in range(IPW // L):
                    for u in range(D // L):
                        plsc.store_scatter(
                            stg.at[b], [colsets[u], row_ids[v]], vecs[n]
                        )
                        n += 1

            pltpu.async_copy(stg.at[b], out_hbm.at[j, :, pl.ds(i0, IPW)], wsem)

    for _ in range(MBUF):
        pltpu.make_async_copy(
            stg.at[0], out_hbm.at[0, :, pl.ds(0, IPW)], wsem
        ).wait()


def kernel(camera_indices, table):
    table_p = jnp.pad(table, ((0, 0), (0, DPAD - D)))
    idx_blk = (
        camera_indices.astype(jnp.int32)
        .reshape(NW, IPW, SEQ)
        .transpose(0, 2, 1)
    )
    out_t = _gather_t_kernel(table_p, idx_blk)
    return out_t.transpose(2, 0, 1)
